# baseline (speedup 1.0000x reference)
import jax
import jax.numpy as jnp
from jax import lax
from jax.experimental import pallas as pl
from jax.experimental.pallas import tpu as pltpu
from jax.experimental.pallas import tpu_sc as plsc

N = 10000
E = 160000
H = 4
NEG = 0.2
EPS = 1e-9

NC = 2
NS = 16
L = 16
HALF = N // NC
ACCR = 5120
NP = NC * ACCR
EPT = E // NS
BLK = 400
NBLK = EPT // BLK
CPB = BLK // L
CPAIR = CPB // 2
RPT = ACCR // NS
RW = 256
BN = 200
GRID = N // BN


SEPT = E // (NC * NS)
SBLKS = [1008, 1008, 1008, 1008, 976]


def _make_score_kernel():
    mesh = plsc.VectorSubcoreMesh(core_axis_name="c", subcore_axis_name="s")
    out_type = [jax.ShapeDtypeStruct(((E + L) * L,), jnp.float32),
                jax.ShapeDtypeStruct((NC * NP, 128), jnp.float32)]
    MB = max(SBLKS)
    scratch = [
        pltpu.VMEM((MB,), jnp.int32),
        pltpu.VMEM((MB,), jnp.int32),
        pltpu.VMEM((MB * L,), jnp.float32),
        pltpu.VMEM((L, 128), jnp.float32),
        pltpu.VMEM((L, 128), jnp.float32),
        pltpu.VMEM((L, 128), jnp.float32),
        pltpu.VMEM((L, 128), jnp.float32),
        pltpu.VMEM((L, 128), jnp.float32),
        pltpu.VMEM((L, 128), jnp.float32),
        pltpu.VMEM((L,), jnp.int32),
        pltpu.VMEM((L,), jnp.int32),
        pltpu.VMEM((L, 128), jnp.float32),
        pltpu.SemaphoreType.DMA,
        pltpu.SemaphoreType.DMA,
        pltpu.SemaphoreType.DMA,
        pltpu.SemaphoreType.DMA,
        pltpu.VMEM_SHARED((NP, 128), jnp.float32),
    ]

    def body(el_h, er_h, src_h, dst_h, ee_h, den_h,
             sblk, dblk, eeblk, bufEA, bufEB, bufRA, bufRB, dbufA, dbufB,
             locA, locB, zeroD, gsemA, gsemB, ssemA, ssemB, accD):
        c = lax.axis_index("c")
        s = lax.axis_index("s")
        wid = s * NC + c
        e_base = wid * SEPT
        zvec = jnp.zeros((L,), jnp.float32)
        iot = lax.iota(jnp.int32, L)
        hmask = iot < H
        rpt2 = NP // NS
        r0 = s * rpt2

        for i in range(L):
            for k in range(128 // L):
                zeroD[i, pl.ds(k * L, L)] = zvec
                dbufA[i, pl.ds(k * L, L)] = zvec
                dbufB[i, pl.ds(k * L, L)] = zvec
        for j in range(rpt2 // L):
            pltpu.sync_copy(zeroD, accD.at[pl.ds(r0 + j * L, L)])
        plsc.subcore_barrier()

        boff = 0
        for bs in SBLKS:
            cpb = bs // L
            cpair = cpb // 2
            pltpu.sync_copy(src_h.at[pl.ds(e_base + boff, bs)],
                            sblk.at[pl.ds(0, bs)])
            pltpu.sync_copy(dst_h.at[pl.ds(e_base + boff, bs)],
                            dblk.at[pl.ds(0, bs)])

            def gstart(chunk, bufE, bufR, gsem):
                sl16 = pl.ds(chunk * L, L)
                pltpu.async_copy(el_h.at[sblk.at[sl16]], bufE, gsem)
                pltpu.async_copy(er_h.at[dblk.at[sl16]], bufR, gsem)

            def gdrain(bufE, bufR, gsem):
                pltpu.make_async_copy(el_h.at[pl.ds(0, L)], bufE,
                                      gsem).wait()
                pltpu.make_async_copy(er_h.at[pl.ds(0, L)], bufR,
                                      gsem).wait()

            def sdrain(dbuf_ref, loc_ref, ssem):
                pltpu.make_async_copy(dbuf_ref, accD.at[loc_ref],
                                      ssem).wait()

            def mkprocess(boff_s):
                def process(chunk, bufE, bufR, dbuf_ref, loc_ref, gsem,
                            ssem):
                    dv = dblk[pl.ds(chunk * L, L)]
                    over = (boff_s + chunk * L) + iot >= SEPT
                    loc = jnp.where(over, N, dv)
                    loc_ref[...] = loc
                    gdrain(bufE, bufR, gsem)
                    for l in range(L):
                        e = bufE[l, pl.ds(0, L)] + bufR[l, pl.ds(0, L)]
                        e = jnp.where(e > 0, e, NEG * e)
                        ee = jnp.where(hmask, jnp.exp(e), 0.0)
                        eeblk[pl.ds((chunk * L + l) * L, L)] = ee
                        dbuf_ref[l, pl.ds(0, L)] = ee
                    pltpu.async_copy(dbuf_ref, accD.at[loc_ref], ssem,
                                     add=True)
                return process

            process = mkprocess(boff)
            gstart(0, bufEA, bufRA, gsemA)

            def pair(j, carry):
                c0 = 2 * j

                @pl.when(j > 0)
                def _():
                    sdrain(dbufB, locB, ssemB)

                gstart(c0 + 1, bufEB, bufRB, gsemB)
                process(c0, bufEA, bufRA, dbufA, locA, gsemA, ssemA)
                process(c0 + 1, bufEB, bufRB, dbufB, locB, gsemB, ssemB)
                sdrain(dbufA, locA, ssemA)

                @pl.when(j < cpair - 1)
                def _():
                    gstart(c0 + 2, bufEA, bufRA, gsemA)

                return carry

            lax.fori_loop(0, cpair, pair, 0)
            sdrain(dbufB, locB, ssemB)
            gstart(cpb - 1, bufEA, bufRA, gsemA)
            process(cpb - 1, bufEA, bufRA, dbufA, locA, gsemA, ssemA)
            sdrain(dbufA, locA, ssemA)

            pltpu.sync_copy(eeblk.at[pl.ds(0, bs * L)],
                            ee_h.at[pl.ds((e_base + boff) * L, bs * L)])
            boff += bs

        plsc.subcore_barrier()
        pltpu.sync_copy(accD.at[pl.ds(r0, rpt2)],
                        den_h.at[pl.ds(c * NP + r0, rpt2)])

    return pl.kernel(body, out_type=out_type, mesh=mesh,
                     scratch_types=scratch)


def _make_agg_kernel(rounds):
    hr = H // rounds
    vh = RW // hr // L
    mesh = plsc.VectorSubcoreMesh(core_axis_name="c", subcore_axis_name="s")
    out_type = [jax.ShapeDtypeStruct((2 * NP, 128), jnp.float32)
                for _ in range(rounds)]
    scratch = [
        pltpu.VMEM((BLK * L,), jnp.float32),
        pltpu.VMEM((BLK,), jnp.int32),
        pltpu.VMEM((BLK,), jnp.int32),
        pltpu.VMEM((L, RW), jnp.float32),
        pltpu.VMEM((L, RW), jnp.float32),
        pltpu.VMEM((L, 128), jnp.float32),
        pltpu.VMEM((L, 128), jnp.float32),
        pltpu.VMEM((L, 128), jnp.float32),
        pltpu.VMEM((L, 128), jnp.float32),
        pltpu.VMEM((L,), jnp.int32),
        pltpu.VMEM((L,), jnp.int32),
        pltpu.VMEM((L,), jnp.int32),
        pltpu.VMEM((L,), jnp.int32),
        pltpu.VMEM((L, 128), jnp.float32),
        pltpu.SemaphoreType.DMA,
        pltpu.SemaphoreType.DMA,
        pltpu.SemaphoreType.DMA,
        pltpu.SemaphoreType.DMA,
        pltpu.VMEM_SHARED((2 * ACCR, 128), jnp.float32),
    ]

    def body(*refs):
        fs_refs = refs[0:rounds]
        ee_h, src_h, dst_h = refs[rounds:rounds + 3]
        outs = refs[rounds + 3:2 * rounds + 3]
        (eeblk, sblk, dblk, rowsA, rowsB, rLA, rRA, rLB, rRB,
         locLA, locRA, locLB, locRB, zero_v,
         gsemA, gsemB, ssemA, ssemB, accA) = refs[2 * rounds + 3:]

        c = lax.axis_index("c")
        s = lax.axis_index("s")
        base = c * HALF
        zvec = jnp.zeros((L,), jnp.float32)
        r0 = s * RPT

        for i in range(L):
            for k in range(128 // L):
                zero_v[i, pl.ds(k * L, L)] = zvec

        def zero_acc():
            for j in range(2 * RPT // L):
                pltpu.sync_copy(zero_v,
                                accA.at[pl.ds(2 * r0 + j * L, L)])

        zero_acc()

        for f in range(rounds):
            fs_h = fs_refs[f]
            plsc.subcore_barrier()

            def blk_body(blk, bcarry):
                e0 = s * EPT + blk * BLK
                pltpu.sync_copy(src_h.at[pl.ds(e0, BLK)], sblk)
                pltpu.sync_copy(dst_h.at[pl.ds(e0, BLK)], dblk)
                pltpu.sync_copy(ee_h.at[pl.ds(e0 * L, BLK * L)], eeblk)

                def gstart(chunk, rows_ref, gsem):
                    sl16 = pl.ds(chunk * L, L)
                    pltpu.async_copy(fs_h.at[sblk.at[sl16]], rows_ref, gsem)

                def gdrain(rows_ref, gsem):
                    pltpu.make_async_copy(fs_h.at[pl.ds(0, L)], rows_ref,
                                          gsem).wait()

                def sdrain(rL, rR, locL, locR, ssem):
                    pltpu.make_async_copy(rL, accA.at[locL], ssem).wait()
                    pltpu.make_async_copy(rR, accA.at[locR], ssem).wait()

                def process(chunk, rows_ref, rL, rR, locL, locR, gsem,
                            ssem):
                    dv = dblk[pl.ds(chunk * L, L)]
                    loc = dv - base
                    oob = (loc < 0) | (loc >= HALF)
                    loc = jnp.where(oob, HALF, loc)
                    locL[...] = loc * 2
                    locR[...] = loc * 2 + 1
                    gdrain(rows_ref, gsem)
                    for l in range(L):
                        eerow = eeblk[pl.ds((chunk * L + l) * L, L)]
                        scs = [eerow[f * hr + h] for h in range(hr)]
                        for k in range(RW // L):
                            sl = pl.ds((k % 8) * L, L)
                            half = rL if k < 8 else rR
                            half[l, sl] = (rows_ref[l, pl.ds(k * L, L)]
                                           * scs[k // vh])
                    pltpu.async_copy(rL, accA.at[locL], ssem, add=True)
                    pltpu.async_copy(rR, accA.at[locR], ssem, add=True)

                gstart(0, rowsA, gsemA)

                def pair(j, carry):
                    c0 = 2 * j

                    @pl.when(j > 0)
                    def _():
                        sdrain(rLB, rRB, locLB, locRB, ssemB)

                    gstart(c0 + 1, rowsB, gsemB)
                    process(c0, rowsA, rLA, rRA, locLA, locRA, gsemA,
                            ssemA)
                    process(c0 + 1, rowsB, rLB, rRB, locLB, locRB, gsemB,
                            ssemB)
                    sdrain(rLA, rRA, locLA, locRA, ssemA)

                    @pl.when(j < CPAIR - 1)
                    def _():
                        gstart(c0 + 2, rowsA, gsemA)

                    return carry

                lax.fori_loop(0, CPAIR, pair, 0)
                sdrain(rLB, rRB, locLB, locRB, ssemB)
                gstart(CPB - 1, rowsA, gsemA)
                process(CPB - 1, rowsA, rLA, rRA, locLA, locRA, gsemA,
                        ssemA)
                sdrain(rLA, rRA, locLA, locRA, ssemA)
                return bcarry

            lax.fori_loop(0, NBLK, blk_body, 0)

            plsc.subcore_barrier()
            pltpu.sync_copy(accA.at[pl.ds(2 * r0, 2 * RPT)],
                            outs[f].at[pl.ds(2 * (c * ACCR + r0),
                                             2 * RPT)])
            if f < rounds - 1:
                zero_acc()

    return pl.kernel(body, out_type=out_type, mesh=mesh,
                     scratch_types=scratch)


_score = _make_score_kernel()
_agg1 = _make_agg_kernel(2)
_agg2 = _make_agg_kernel(1)


def _k1_body(x_ref, ws_ref, wd_ref, al_ref, ar_ref,
             fa_ref, fb_ref, el_ref, er_ref):
    xb = x_ref[...]
    g = (lax.broadcasted_iota(jnp.int32, (4 * 128, 128), 0) // 128
         == lax.broadcasted_iota(jnp.int32, (4 * 128, 128), 1)
         ).astype(jnp.float32)
    fs = jnp.dot(xb, ws_ref[...], preferred_element_type=jnp.float32)
    fa_ref[...] = fs[:, 0:256]
    fb_ref[...] = fs[:, 256:512]
    el_ref[...] = jnp.dot(fs * al_ref[...], g,
                          preferred_element_type=jnp.float32)
    fd = jnp.dot(xb, wd_ref[...], preferred_element_type=jnp.float32)
    er_ref[...] = jnp.dot(fd * ar_ref[...], g,
                          preferred_element_type=jnp.float32)


def _k3_body(ra_ref, rb_ref, dna_ref, dnb_ref, ba_ref, bb_ref, wsa_ref,
             wsb_ref, wda_ref, wdb_ref, al_ref, ar_ref, fs_ref, el_ref,
             er_ref):
    d4 = dna_ref[...] + dnb_ref[...]
    rr = lax.broadcasted_iota(jnp.int32, (H, 256), 0)
    ch = lax.broadcasted_iota(jnp.int32, (H, 256), 1) // 128
    sa = jnp.dot(d4, (ch == rr).astype(jnp.float32),
                 preferred_element_type=jnp.float32) + EPS
    sb = jnp.dot(d4, (ch + 2 == rr).astype(jnp.float32),
                 preferred_element_type=jnp.float32) + EPS
    ha = jnp.maximum(ra_ref[...] / sa + ba_ref[...], 0.0)
    hb = jnp.maximum(rb_ref[...] / sb + bb_ref[...], 0.0)
    fs2 = (jnp.dot(ha, wsa_ref[...], preferred_element_type=jnp.float32)
           + jnp.dot(hb, wsb_ref[...], preferred_element_type=jnp.float32))
    fs_ref[...] = fs2
    g2 = (lax.broadcasted_iota(jnp.int32, (256, 128), 0) // 64
          == lax.broadcasted_iota(jnp.int32, (256, 128), 1)
          ).astype(jnp.float32)
    el_ref[...] = jnp.dot(fs2 * al_ref[...], g2,
                          preferred_element_type=jnp.float32)
    fd2 = (jnp.dot(ha, wda_ref[...], preferred_element_type=jnp.float32)
           + jnp.dot(hb, wdb_ref[...], preferred_element_type=jnp.float32))
    er_ref[...] = jnp.dot(fd2 * ar_ref[...], g2,
                          preferred_element_type=jnp.float32)


def _k5_body(rc_ref, dna_ref, dnb_ref, b2_ref, o_ref):
    d4 = dna_ref[...] + dnb_ref[...]
    rr = lax.broadcasted_iota(jnp.int32, (H, 256), 0)
    ch = lax.broadcasted_iota(jnp.int32, (H, 256), 1) // 64
    sc = jnp.dot(d4, (ch == rr).astype(jnp.float32),
                 preferred_element_type=jnp.float32) + EPS
    t = rc_ref[...] / sc + b2_ref[...]
    m = ((lax.broadcasted_iota(jnp.int32, (256, 64), 0) % 64
          == lax.broadcasted_iota(jnp.int32, (256, 64), 1))
         ).astype(jnp.float32) * 0.25
    o_ref[...] = jnp.dot(t, m, preferred_element_type=jnp.float32)


def _row_spec(w):
    return pl.BlockSpec((BN, w), lambda i: (i, 0))


def _full_spec(shape):
    return pl.BlockSpec(shape, lambda i: tuple(0 for _ in shape))


_k1 = pl.pallas_call(
    _k1_body,
    grid=(GRID,),
    in_specs=[_row_spec(256), _full_spec((256, 512)), _full_spec((256, 512)),
              _full_spec((1, 512)), _full_spec((1, 512))],
    out_specs=[_row_spec(RW), _row_spec(RW), _row_spec(128), _row_spec(128)],
    out_shape=[jax.ShapeDtypeStruct((N, RW), jnp.float32),
               jax.ShapeDtypeStruct((N, RW), jnp.float32),
               jax.ShapeDtypeStruct((N, 128), jnp.float32),
               jax.ShapeDtypeStruct((N, 128), jnp.float32)],
)

_k3 = pl.pallas_call(
    _k3_body,
    grid=(GRID,),
    in_specs=[_row_spec(RW), _row_spec(RW), _row_spec(H), _row_spec(H),
              _full_spec((1, RW)), _full_spec((1, RW)),
              _full_spec((RW, RW)), _full_spec((RW, RW)),
              _full_spec((RW, RW)), _full_spec((RW, RW)),
              _full_spec((1, RW)), _full_spec((1, RW))],
    out_specs=[_row_spec(RW), _row_spec(128), _row_spec(128)],
    out_shape=[jax.ShapeDtypeStruct((N, RW), jnp.float32),
               jax.ShapeDtypeStruct((N, 128), jnp.float32),
               jax.ShapeDtypeStruct((N, 128), jnp.float32)],
)

_k5 = pl.pallas_call(
    _k5_body,
    grid=(GRID,),
    in_specs=[_row_spec(RW), _row_spec(H), _row_spec(H),
              _full_spec((1, RW))],
    out_specs=_row_spec(64),
    out_shape=jax.ShapeDtypeStruct((N, 64), jnp.float32),
)


def _unpad(a):
    return jnp.concatenate([a[:HALF], a[ACCR:ACCR + HALF]], axis=0)


def _edge_phase(fs_parts, el, er, src, dst, agg):
    pad = jnp.zeros((L,), jnp.int32)
    srcp = jnp.concatenate([src, pad])
    dstp = jnp.concatenate([dst, pad])
    ee, den = _score(el, er, srcp, dstp)
    raws = agg(*fs_parts, ee, src, dst)
    if not isinstance(raws, (list, tuple)):
        raws = (raws,)
    dna = den[0:N, :H]
    dnb = den[NP:NP + N, :H]
    return [_unpad(r.reshape(NP, RW)) for r in raws], (dna, dnb)


def kernel(x, edge_index1, edge_index2, W_src1, W_dst1, attn_l1, attn_r1,
           bias1, W_src2, W_dst2, attn_l2, attn_r2, bias2):
    al1 = attn_l1.reshape(1, H * 128)
    ar1 = attn_r1.reshape(1, H * 128)
    fa1, fb1, el1, er1 = _k1(x, W_src1, W_dst1, al1, ar1)

    (ra, rb), (dn1a, dn1b) = _edge_phase((fa1, fb1), el1, er1,
                                         edge_index1[0], edge_index1[1],
                                         _agg1)

    ba = bias1[:RW].reshape(1, RW)
    bb = bias1[RW:].reshape(1, RW)
    al2 = attn_l2.reshape(1, H * 64)
    ar2 = attn_r2.reshape(1, H * 64)
    fs2, el2, er2 = _k3(ra, rb, dn1a, dn1b, ba, bb,
                        W_src2[:RW], W_src2[RW:],
                        W_dst2[:RW], W_dst2[RW:], al2, ar2)

    (rc,), (dn2a, dn2b) = _edge_phase((fs2,), el2, er2,
                                      edge_index2[0], edge_index2[1],
                                      _agg2)

    return _k5(rc, dn2a, dn2b, bias2.reshape(1, RW))


# speedup vs baseline: 1.1119x; 1.1119x over previous
import jax
import jax.numpy as jnp
from jax import lax
from jax.experimental import pallas as pl
from jax.experimental.pallas import tpu as pltpu
from jax.experimental.pallas import tpu_sc as plsc

N = 10000
E = 160000
H = 4
NEG = 0.2
EPS = 1e-9

NC = 2
NS = 16
L = 16
HALF = N // NC
ACCR = 5120
NP = NC * ACCR
EPT = E // NS
BLK = 400
NBLK = EPT // BLK
CPB = BLK // L
CPAIR = CPB // 2
RPT = ACCR // NS
RW = 256
BN = 200
GRID = N // BN


SEPT = E // (NC * NS)
SBLKS = [1008, 1008, 1008, 1008, 976]


def _make_score_kernel():
    mesh = plsc.VectorSubcoreMesh(core_axis_name="c", subcore_axis_name="s")
    out_type = [jax.ShapeDtypeStruct(((E + L) * L,), jnp.float32),
                jax.ShapeDtypeStruct((NC * NP, 128), jnp.float32)]
    MB = max(SBLKS)
    scratch = [
        pltpu.VMEM((MB,), jnp.int32),
        pltpu.VMEM((MB,), jnp.int32),
        pltpu.VMEM((MB * L,), jnp.float32),
        pltpu.VMEM((L, 128), jnp.float32),
        pltpu.VMEM((L, 128), jnp.float32),
        pltpu.VMEM((L, 128), jnp.float32),
        pltpu.VMEM((L, 128), jnp.float32),
        pltpu.VMEM((L, 128), jnp.float32),
        pltpu.VMEM((L, 128), jnp.float32),
        pltpu.VMEM((L,), jnp.int32),
        pltpu.VMEM((L,), jnp.int32),
        pltpu.VMEM((L, 128), jnp.float32),
        pltpu.SemaphoreType.DMA,
        pltpu.SemaphoreType.DMA,
        pltpu.SemaphoreType.DMA,
        pltpu.SemaphoreType.DMA,
        pltpu.VMEM_SHARED((NP, 128), jnp.float32),
    ]

    def body(el_h, er_h, src_h, dst_h, ee_h, den_h,
             sblk, dblk, eeblk, bufEA, bufEB, bufRA, bufRB, dbufA, dbufB,
             locA, locB, zeroD, gsemA, gsemB, ssemA, ssemB, accD):
        c = lax.axis_index("c")
        s = lax.axis_index("s")
        wid = s * NC + c
        e_base = wid * SEPT
        zvec = jnp.zeros((L,), jnp.float32)
        iot = lax.iota(jnp.int32, L)
        hmask = iot < H
        rpt2 = NP // NS
        r0 = s * rpt2

        for i in range(L):
            for k in range(128 // L):
                zeroD[i, pl.ds(k * L, L)] = zvec
                dbufA[i, pl.ds(k * L, L)] = zvec
                dbufB[i, pl.ds(k * L, L)] = zvec
        for j in range(rpt2 // L):
            pltpu.sync_copy(zeroD, accD.at[pl.ds(r0 + j * L, L)])
        plsc.subcore_barrier()

        boff = 0
        for bs in SBLKS:
            cpb = bs // L
            cpair = cpb // 2
            pltpu.sync_copy(src_h.at[pl.ds(e_base + boff, bs)],
                            sblk.at[pl.ds(0, bs)])
            pltpu.sync_copy(dst_h.at[pl.ds(e_base + boff, bs)],
                            dblk.at[pl.ds(0, bs)])

            def gstart(chunk, bufE, bufR, gsem):
                sl16 = pl.ds(chunk * L, L)
                pltpu.async_copy(el_h.at[sblk.at[sl16]], bufE, gsem)
                pltpu.async_copy(er_h.at[dblk.at[sl16]], bufR, gsem)

            def gdrain(bufE, bufR, gsem):
                pltpu.make_async_copy(el_h.at[pl.ds(0, L)], bufE,
                                      gsem).wait()
                pltpu.make_async_copy(er_h.at[pl.ds(0, L)], bufR,
                                      gsem).wait()

            def sdrain(dbuf_ref, loc_ref, ssem):
                pltpu.make_async_copy(dbuf_ref, accD.at[loc_ref],
                                      ssem).wait()

            def mkprocess(boff_s):
                def process(chunk, bufE, bufR, dbuf_ref, loc_ref, gsem,
                            ssem):
                    dv = dblk[pl.ds(chunk * L, L)]
                    over = (boff_s + chunk * L) + iot >= SEPT
                    loc = jnp.where(over, N, dv)
                    loc_ref[...] = loc
                    gdrain(bufE, bufR, gsem)
                    for l in range(L):
                        e = bufE[l, pl.ds(0, L)] + bufR[l, pl.ds(0, L)]
                        e = jnp.where(e > 0, e, NEG * e)
                        ee = jnp.where(hmask, jnp.exp(e), 0.0)
                        eeblk[pl.ds((chunk * L + l) * L, L)] = ee
                        dbuf_ref[l, pl.ds(0, L)] = ee
                    pltpu.async_copy(dbuf_ref, accD.at[loc_ref], ssem,
                                     add=True)
                return process

            process = mkprocess(boff)
            gstart(0, bufEA, bufRA, gsemA)

            def pair(j, carry):
                c0 = 2 * j

                @pl.when(j > 0)
                def _():
                    sdrain(dbufB, locB, ssemB)

                gstart(c0 + 1, bufEB, bufRB, gsemB)
                process(c0, bufEA, bufRA, dbufA, locA, gsemA, ssemA)
                process(c0 + 1, bufEB, bufRB, dbufB, locB, gsemB, ssemB)
                sdrain(dbufA, locA, ssemA)

                @pl.when(j < cpair - 1)
                def _():
                    gstart(c0 + 2, bufEA, bufRA, gsemA)

                return carry

            lax.fori_loop(0, cpair, pair, 0)
            sdrain(dbufB, locB, ssemB)
            gstart(cpb - 1, bufEA, bufRA, gsemA)
            process(cpb - 1, bufEA, bufRA, dbufA, locA, gsemA, ssemA)
            sdrain(dbufA, locA, ssemA)

            pltpu.sync_copy(eeblk.at[pl.ds(0, bs * L)],
                            ee_h.at[pl.ds((e_base + boff) * L, bs * L)])
            boff += bs

        plsc.subcore_barrier()
        pltpu.sync_copy(accD.at[pl.ds(r0, rpt2)],
                        den_h.at[pl.ds(c * NP + r0, rpt2)])

    return pl.kernel(body, out_type=out_type, mesh=mesh,
                     scratch_types=scratch)


def _make_agg_kernel(rounds):
    hr = H // rounds
    vh = RW // hr // L
    mesh = plsc.VectorSubcoreMesh(core_axis_name="c", subcore_axis_name="s")
    out_type = [jax.ShapeDtypeStruct((2 * NP, 128), jnp.float32)
                for _ in range(rounds)]
    scratch = [
        pltpu.VMEM((BLK * L,), jnp.float32),
        pltpu.VMEM((BLK,), jnp.int32),
        pltpu.VMEM((BLK,), jnp.int32),
        pltpu.VMEM((L, RW), jnp.float32),
        pltpu.VMEM((L, RW), jnp.float32),
        pltpu.VMEM((L, RW), jnp.float32),
        pltpu.VMEM((L, 128), jnp.float32),
        pltpu.VMEM((L, 128), jnp.float32),
        pltpu.VMEM((L, 128), jnp.float32),
        pltpu.VMEM((L, 128), jnp.float32),
        pltpu.VMEM((L, 128), jnp.float32),
        pltpu.VMEM((L, 128), jnp.float32),
        pltpu.VMEM((L,), jnp.int32),
        pltpu.VMEM((L,), jnp.int32),
        pltpu.VMEM((L,), jnp.int32),
        pltpu.VMEM((L,), jnp.int32),
        pltpu.VMEM((L,), jnp.int32),
        pltpu.VMEM((L,), jnp.int32),
        pltpu.VMEM((L, 128), jnp.float32),
        pltpu.SemaphoreType.DMA,
        pltpu.SemaphoreType.DMA,
        pltpu.SemaphoreType.DMA,
        pltpu.SemaphoreType.DMA,
        pltpu.SemaphoreType.DMA,
        pltpu.SemaphoreType.DMA,
        pltpu.VMEM_SHARED((2 * ACCR, 128), jnp.float32),
    ]

    def body(*refs):
        fs_refs = refs[0:rounds]
        ee_h, src_h, dst_h = refs[rounds:rounds + 3]
        outs = refs[rounds + 3:2 * rounds + 3]
        (eeblk, sblk, dblk, rowsA, rowsB, rowsC, rLA, rRA, rLB, rRB,
         rLC, rRC, locLA, locRA, locLB, locRB, locLC, locRC, zero_v,
         gsemA, gsemB, gsemC, ssemA, ssemB, ssemC, accA) = \
            refs[2 * rounds + 3:]

        c = lax.axis_index("c")
        s = lax.axis_index("s")
        base = c * HALF
        zvec = jnp.zeros((L,), jnp.float32)
        r0 = s * RPT

        for i in range(L):
            for k in range(128 // L):
                zero_v[i, pl.ds(k * L, L)] = zvec

        def zero_acc():
            for j in range(2 * RPT // L):
                pltpu.sync_copy(zero_v,
                                accA.at[pl.ds(2 * r0 + j * L, L)])

        zero_acc()

        for f in range(rounds):
            fs_h = fs_refs[f]
            plsc.subcore_barrier()

            def blk_body(blk, bcarry):
                e0 = s * EPT + blk * BLK
                pltpu.sync_copy(src_h.at[pl.ds(e0, BLK)], sblk)
                pltpu.sync_copy(dst_h.at[pl.ds(e0, BLK)], dblk)
                pltpu.sync_copy(ee_h.at[pl.ds(e0 * L, BLK * L)], eeblk)

                def gstart(chunk, rows_ref, gsem):
                    sl16 = pl.ds(chunk * L, L)
                    pltpu.async_copy(fs_h.at[sblk.at[sl16]], rows_ref, gsem)

                def gdrain(rows_ref, gsem):
                    pltpu.make_async_copy(fs_h.at[pl.ds(0, L)], rows_ref,
                                          gsem).wait()

                def sdrain(rL, rR, locL, locR, ssem):
                    pltpu.make_async_copy(rL, accA.at[locL], ssem).wait()
                    pltpu.make_async_copy(rR, accA.at[locR], ssem).wait()

                def process(chunk, rows_ref, rL, rR, locL, locR, gsem,
                            ssem):
                    dv = dblk[pl.ds(chunk * L, L)]
                    loc = dv - base
                    oob = (loc < 0) | (loc >= HALF)
                    loc = jnp.where(oob, HALF, loc)
                    locL[...] = loc * 2
                    locR[...] = loc * 2 + 1
                    gdrain(rows_ref, gsem)
                    for l in range(L):
                        eerow = eeblk[pl.ds((chunk * L + l) * L, L)]
                        scs = [eerow[f * hr + h] for h in range(hr)]
                        for k in range(RW // L):
                            sl = pl.ds((k % 8) * L, L)
                            half = rL if k < 8 else rR
                            half[l, sl] = (rows_ref[l, pl.ds(k * L, L)]
                                           * scs[k // vh])
                    pltpu.async_copy(rL, accA.at[locL], ssem, add=True)
                    pltpu.async_copy(rR, accA.at[locR], ssem, add=True)

                NT = (CPB - 1) // 3

                gstart(0, rowsA, gsemA)
                gstart(1, rowsB, gsemB)

                def triple(j, carry):
                    c0 = 3 * j

                    @pl.when(j > 0)
                    def _():
                        sdrain(rLC, rRC, locLC, locRC, ssemC)

                    gstart(c0 + 2, rowsC, gsemC)
                    process(c0, rowsA, rLA, rRA, locLA, locRA, gsemA,
                            ssemA)
                    process(c0 + 1, rowsB, rLB, rRB, locLB, locRB, gsemB,
                            ssemB)
                    sdrain(rLA, rRA, locLA, locRA, ssemA)
                    gstart(c0 + 3, rowsA, gsemA)
                    process(c0 + 2, rowsC, rLC, rRC, locLC, locRC, gsemC,
                            ssemC)
                    sdrain(rLB, rRB, locLB, locRB, ssemB)

                    @pl.when(j < NT - 1)
                    def _():
                        gstart(c0 + 4, rowsB, gsemB)

                    return carry

                lax.fori_loop(0, NT, triple, 0)
                sdrain(rLC, rRC, locLC, locRC, ssemC)
                process(CPB - 1, rowsA, rLA, rRA, locLA, locRA, gsemA,
                        ssemA)
                sdrain(rLA, rRA, locLA, locRA, ssemA)
                return bcarry

            lax.fori_loop(0, NBLK, blk_body, 0)

            plsc.subcore_barrier()
            pltpu.sync_copy(accA.at[pl.ds(2 * r0, 2 * RPT)],
                            outs[f].at[pl.ds(2 * (c * ACCR + r0),
                                             2 * RPT)])
            if f < rounds - 1:
                zero_acc()

    return pl.kernel(body, out_type=out_type, mesh=mesh,
                     scratch_types=scratch)


_score = _make_score_kernel()
_agg1 = _make_agg_kernel(2)
_agg2 = _make_agg_kernel(1)


def _k1_body(x_ref, ws_ref, wd_ref, al_ref, ar_ref,
             fa_ref, fb_ref, el_ref, er_ref):
    xb = x_ref[...]
    g = (lax.broadcasted_iota(jnp.int32, (4 * 128, 128), 0) // 128
         == lax.broadcasted_iota(jnp.int32, (4 * 128, 128), 1)
         ).astype(jnp.float32)
    fs = jnp.dot(xb, ws_ref[...], preferred_element_type=jnp.float32)
    fa_ref[...] = fs[:, 0:256]
    fb_ref[...] = fs[:, 256:512]
    el_ref[...] = jnp.dot(fs * al_ref[...], g,
                          preferred_element_type=jnp.float32)
    fd = jnp.dot(xb, wd_ref[...], preferred_element_type=jnp.float32)
    er_ref[...] = jnp.dot(fd * ar_ref[...], g,
                          preferred_element_type=jnp.float32)


def _k3_body(ra_ref, rb_ref, dna_ref, dnb_ref, ba_ref, bb_ref, wsa_ref,
             wsb_ref, wda_ref, wdb_ref, al_ref, ar_ref, fs_ref, el_ref,
             er_ref):
    d4 = dna_ref[...] + dnb_ref[...]
    rr = lax.broadcasted_iota(jnp.int32, (H, 256), 0)
    ch = lax.broadcasted_iota(jnp.int32, (H, 256), 1) // 128
    sa = jnp.dot(d4, (ch == rr).astype(jnp.float32),
                 preferred_element_type=jnp.float32) + EPS
    sb = jnp.dot(d4, (ch + 2 == rr).astype(jnp.float32),
                 preferred_element_type=jnp.float32) + EPS
    ha = jnp.maximum(ra_ref[...] / sa + ba_ref[...], 0.0)
    hb = jnp.maximum(rb_ref[...] / sb + bb_ref[...], 0.0)
    fs2 = (jnp.dot(ha, wsa_ref[...], preferred_element_type=jnp.float32)
           + jnp.dot(hb, wsb_ref[...], preferred_element_type=jnp.float32))
    fs_ref[...] = fs2
    g2 = (lax.broadcasted_iota(jnp.int32, (256, 128), 0) // 64
          == lax.broadcasted_iota(jnp.int32, (256, 128), 1)
          ).astype(jnp.float32)
    el_ref[...] = jnp.dot(fs2 * al_ref[...], g2,
                          preferred_element_type=jnp.float32)
    fd2 = (jnp.dot(ha, wda_ref[...], preferred_element_type=jnp.float32)
           + jnp.dot(hb, wdb_ref[...], preferred_element_type=jnp.float32))
    er_ref[...] = jnp.dot(fd2 * ar_ref[...], g2,
                          preferred_element_type=jnp.float32)


def _k5_body(rc_ref, dna_ref, dnb_ref, b2_ref, o_ref):
    d4 = dna_ref[...] + dnb_ref[...]
    rr = lax.broadcasted_iota(jnp.int32, (H, 256), 0)
    ch = lax.broadcasted_iota(jnp.int32, (H, 256), 1) // 64
    sc = jnp.dot(d4, (ch == rr).astype(jnp.float32),
                 preferred_element_type=jnp.float32) + EPS
    t = rc_ref[...] / sc + b2_ref[...]
    m = ((lax.broadcasted_iota(jnp.int32, (256, 64), 0) % 64
          == lax.broadcasted_iota(jnp.int32, (256, 64), 1))
         ).astype(jnp.float32) * 0.25
    o_ref[...] = jnp.dot(t, m, preferred_element_type=jnp.float32)


def _row_spec(w):
    return pl.BlockSpec((BN, w), lambda i: (i, 0))


def _full_spec(shape):
    return pl.BlockSpec(shape, lambda i: tuple(0 for _ in shape))


_k1 = pl.pallas_call(
    _k1_body,
    grid=(GRID,),
    in_specs=[_row_spec(256), _full_spec((256, 512)), _full_spec((256, 512)),
              _full_spec((1, 512)), _full_spec((1, 512))],
    out_specs=[_row_spec(RW), _row_spec(RW), _row_spec(128), _row_spec(128)],
    out_shape=[jax.ShapeDtypeStruct((N, RW), jnp.float32),
               jax.ShapeDtypeStruct((N, RW), jnp.float32),
               jax.ShapeDtypeStruct((N, 128), jnp.float32),
               jax.ShapeDtypeStruct((N, 128), jnp.float32)],
)

_k3 = pl.pallas_call(
    _k3_body,
    grid=(GRID,),
    in_specs=[_row_spec(RW), _row_spec(RW), _row_spec(H), _row_spec(H),
              _full_spec((1, RW)), _full_spec((1, RW)),
              _full_spec((RW, RW)), _full_spec((RW, RW)),
              _full_spec((RW, RW)), _full_spec((RW, RW)),
              _full_spec((1, RW)), _full_spec((1, RW))],
    out_specs=[_row_spec(RW), _row_spec(128), _row_spec(128)],
    out_shape=[jax.ShapeDtypeStruct((N, RW), jnp.float32),
               jax.ShapeDtypeStruct((N, 128), jnp.float32),
               jax.ShapeDtypeStruct((N, 128), jnp.float32)],
)

_k5 = pl.pallas_call(
    _k5_body,
    grid=(GRID,),
    in_specs=[_row_spec(RW), _row_spec(H), _row_spec(H),
              _full_spec((1, RW))],
    out_specs=_row_spec(64),
    out_shape=jax.ShapeDtypeStruct((N, 64), jnp.float32),
)


def _unpad(a):
    return jnp.concatenate([a[:HALF], a[ACCR:ACCR + HALF]], axis=0)


def _edge_phase(fs_parts, el, er, src, dst, agg):
    pad = jnp.zeros((L,), jnp.int32)
    srcp = jnp.concatenate([src, pad])
    dstp = jnp.concatenate([dst, pad])
    ee, den = _score(el, er, srcp, dstp)
    raws = agg(*fs_parts, ee, src, dst)
    if not isinstance(raws, (list, tuple)):
        raws = (raws,)
    dna = den[0:N, :H]
    dnb = den[NP:NP + N, :H]
    return [_unpad(r.reshape(NP, RW)) for r in raws], (dna, dnb)


def kernel(x, edge_index1, edge_index2, W_src1, W_dst1, attn_l1, attn_r1,
           bias1, W_src2, W_dst2, attn_l2, attn_r2, bias2):
    al1 = attn_l1.reshape(1, H * 128)
    ar1 = attn_r1.reshape(1, H * 128)
    fa1, fb1, el1, er1 = _k1(x, W_src1, W_dst1, al1, ar1)

    (ra, rb), (dn1a, dn1b) = _edge_phase((fa1, fb1), el1, er1,
                                         edge_index1[0], edge_index1[1],
                                         _agg1)

    ba = bias1[:RW].reshape(1, RW)
    bb = bias1[RW:].reshape(1, RW)
    al2 = attn_l2.reshape(1, H * 64)
    ar2 = attn_r2.reshape(1, H * 64)
    fs2, el2, er2 = _k3(ra, rb, dn1a, dn1b, ba, bb,
                        W_src2[:RW], W_src2[RW:],
                        W_dst2[:RW], W_dst2[RW:], al2, ar2)

    (rc,), (dn2a, dn2b) = _edge_phase((fs2,), el2, er2,
                                      edge_index2[0], edge_index2[1],
                                      _agg2)

    return _k5(rc, dn2a, dn2b, bias2.reshape(1, RW))


# speedup vs baseline: 1.1842x; 1.0650x over previous
import jax
import jax.numpy as jnp
from jax import lax
from jax.experimental import pallas as pl
from jax.experimental.pallas import tpu as pltpu
from jax.experimental.pallas import tpu_sc as plsc

N = 10000
E = 160000
H = 4
NEG = 0.2
EPS = 1e-9

NC = 2
NS = 16
L = 16
HALF = N // NC
ACCR = 5120
NP = NC * ACCR
EPT = E // NS
BLK = 400
NBLK = EPT // BLK
CPB = BLK // L
CPAIR = CPB // 2
RPT = ACCR // NS
RW = 256
BN = 200
GRID = N // BN


SEPT = E // (NC * NS)
SBLKS = [1008, 1008, 1008, 1008, 976]


def _make_score_kernel():
    mesh = plsc.VectorSubcoreMesh(core_axis_name="c", subcore_axis_name="s")
    out_type = [jax.ShapeDtypeStruct(((E + L) * L,), jnp.float32),
                jax.ShapeDtypeStruct((NC * NP, 128), jnp.float32)]
    MB = max(SBLKS)
    scratch = [
        pltpu.VMEM((MB,), jnp.int32),
        pltpu.VMEM((MB,), jnp.int32),
        pltpu.VMEM((MB * L,), jnp.float32),
        pltpu.VMEM((L, 128), jnp.float32),
        pltpu.VMEM((L, 128), jnp.float32),
        pltpu.VMEM((L, 128), jnp.float32),
        pltpu.VMEM((L, 128), jnp.float32),
        pltpu.VMEM((L, 128), jnp.float32),
        pltpu.VMEM((L, 128), jnp.float32),
        pltpu.VMEM((L, 128), jnp.float32),
        pltpu.VMEM((L, 128), jnp.float32),
        pltpu.VMEM((L, 128), jnp.float32),
        pltpu.VMEM((L,), jnp.int32),
        pltpu.VMEM((L,), jnp.int32),
        pltpu.VMEM((L,), jnp.int32),
        pltpu.VMEM((L, 128), jnp.float32),
        pltpu.SemaphoreType.DMA,
        pltpu.SemaphoreType.DMA,
        pltpu.SemaphoreType.DMA,
        pltpu.SemaphoreType.DMA,
        pltpu.SemaphoreType.DMA,
        pltpu.SemaphoreType.DMA,
        pltpu.VMEM_SHARED((NP, 128), jnp.float32),
    ]

    def body(el_h, er_h, src_h, dst_h, ee_h, den_h,
             sblk, dblk, eeblk, bufEA, bufEB, bufEC, bufRA, bufRB, bufRC,
             dbufA, dbufB, dbufC, locA, locB, locC, zeroD,
             gsemA, gsemB, gsemC, ssemA, ssemB, ssemC, accD):
        c = lax.axis_index("c")
        s = lax.axis_index("s")
        wid = s * NC + c
        e_base = wid * SEPT
        zvec = jnp.zeros((L,), jnp.float32)
        iot = lax.iota(jnp.int32, L)
        hmask = iot < H
        rpt2 = NP // NS
        r0 = s * rpt2

        for i in range(L):
            for k in range(128 // L):
                zeroD[i, pl.ds(k * L, L)] = zvec
                dbufA[i, pl.ds(k * L, L)] = zvec
                dbufB[i, pl.ds(k * L, L)] = zvec
                dbufC[i, pl.ds(k * L, L)] = zvec
        for j in range(rpt2 // L):
            pltpu.sync_copy(zeroD, accD.at[pl.ds(r0 + j * L, L)])
        plsc.subcore_barrier()

        boff = 0
        for bs in SBLKS:
            cpb = bs // L
            cpair = cpb // 2
            pltpu.sync_copy(src_h.at[pl.ds(e_base + boff, bs)],
                            sblk.at[pl.ds(0, bs)])
            pltpu.sync_copy(dst_h.at[pl.ds(e_base + boff, bs)],
                            dblk.at[pl.ds(0, bs)])

            def gstart(chunk, bufE, bufR, gsem):
                sl16 = pl.ds(chunk * L, L)
                pltpu.async_copy(el_h.at[sblk.at[sl16]], bufE, gsem)
                pltpu.async_copy(er_h.at[dblk.at[sl16]], bufR, gsem)

            def gdrain(bufE, bufR, gsem):
                pltpu.make_async_copy(el_h.at[pl.ds(0, L)], bufE,
                                      gsem).wait()
                pltpu.make_async_copy(er_h.at[pl.ds(0, L)], bufR,
                                      gsem).wait()

            def sdrain(dbuf_ref, loc_ref, ssem):
                pltpu.make_async_copy(dbuf_ref, accD.at[loc_ref],
                                      ssem).wait()

            def mkprocess(boff_s):
                def process(chunk, bufE, bufR, dbuf_ref, loc_ref, gsem,
                            ssem):
                    dv = dblk[pl.ds(chunk * L, L)]
                    over = (boff_s + chunk * L) + iot >= SEPT
                    loc = jnp.where(over, N, dv)
                    loc_ref[...] = loc
                    gdrain(bufE, bufR, gsem)
                    for l in range(L):
                        e = bufE[l, pl.ds(0, L)] + bufR[l, pl.ds(0, L)]
                        e = jnp.where(e > 0, e, NEG * e)
                        ee = jnp.where(hmask, jnp.exp(e), 0.0)
                        eeblk[pl.ds((chunk * L + l) * L, L)] = ee
                        dbuf_ref[l, pl.ds(0, L)] = ee
                    pltpu.async_copy(dbuf_ref, accD.at[loc_ref], ssem,
                                     add=True)
                return process

            process = mkprocess(boff)
            has_tail = cpb % 3 == 1
            nt = cpb // 3
            gstart(0, bufEA, bufRA, gsemA)
            gstart(1, bufEB, bufRB, gsemB)

            def triple(j, carry):
                c0 = 3 * j

                @pl.when(j > 0)
                def _():
                    sdrain(dbufC, locC, ssemC)

                gstart(c0 + 2, bufEC, bufRC, gsemC)
                process(c0, bufEA, bufRA, dbufA, locA, gsemA, ssemA)
                process(c0 + 1, bufEB, bufRB, dbufB, locB, gsemB, ssemB)
                sdrain(dbufA, locA, ssemA)

                @pl.when(c0 + 3 < cpb)
                def _():
                    gstart(c0 + 3, bufEA, bufRA, gsemA)

                process(c0 + 2, bufEC, bufRC, dbufC, locC, gsemC, ssemC)
                sdrain(dbufB, locB, ssemB)

                @pl.when(c0 + 4 < cpb)
                def _():
                    gstart(c0 + 4, bufEB, bufRB, gsemB)

                return carry

            lax.fori_loop(0, nt, triple, 0)
            sdrain(dbufC, locC, ssemC)
            if has_tail:
                process(cpb - 1, bufEA, bufRA, dbufA, locA, gsemA, ssemA)
                sdrain(dbufA, locA, ssemA)

            pltpu.sync_copy(eeblk.at[pl.ds(0, bs * L)],
                            ee_h.at[pl.ds((e_base + boff) * L, bs * L)])
            boff += bs

        plsc.subcore_barrier()
        pltpu.sync_copy(accD.at[pl.ds(r0, rpt2)],
                        den_h.at[pl.ds(c * NP + r0, rpt2)])

    return pl.kernel(body, out_type=out_type, mesh=mesh,
                     scratch_types=scratch)


def _make_agg_kernel(rounds):
    hr = H // rounds
    vh = RW // hr // L
    mesh = plsc.VectorSubcoreMesh(core_axis_name="c", subcore_axis_name="s")
    out_type = [jax.ShapeDtypeStruct((2 * NP, 128), jnp.float32)
                for _ in range(rounds)]
    scratch = [
        pltpu.VMEM((BLK * L,), jnp.float32),
        pltpu.VMEM((BLK,), jnp.int32),
        pltpu.VMEM((BLK,), jnp.int32),
        pltpu.VMEM((L, RW), jnp.float32),
        pltpu.VMEM((L, RW), jnp.float32),
        pltpu.VMEM((L, RW), jnp.float32),
        pltpu.VMEM((L, 128), jnp.float32),
        pltpu.VMEM((L, 128), jnp.float32),
        pltpu.VMEM((L, 128), jnp.float32),
        pltpu.VMEM((L, 128), jnp.float32),
        pltpu.VMEM((L, 128), jnp.float32),
        pltpu.VMEM((L, 128), jnp.float32),
        pltpu.VMEM((L,), jnp.int32),
        pltpu.VMEM((L,), jnp.int32),
        pltpu.VMEM((L,), jnp.int32),
        pltpu.VMEM((L,), jnp.int32),
        pltpu.VMEM((L,), jnp.int32),
        pltpu.VMEM((L,), jnp.int32),
        pltpu.VMEM((L, 128), jnp.float32),
        pltpu.SemaphoreType.DMA,
        pltpu.SemaphoreType.DMA,
        pltpu.SemaphoreType.DMA,
        pltpu.SemaphoreType.DMA,
        pltpu.SemaphoreType.DMA,
        pltpu.SemaphoreType.DMA,
        pltpu.VMEM_SHARED((2 * ACCR, 128), jnp.float32),
    ]

    def body(*refs):
        fs_refs = refs[0:rounds]
        ee_h, src_h, dst_h = refs[rounds:rounds + 3]
        outs = refs[rounds + 3:2 * rounds + 3]
        (eeblk, sblk, dblk, rowsA, rowsB, rowsC, rLA, rRA, rLB, rRB,
         rLC, rRC, locLA, locRA, locLB, locRB, locLC, locRC, zero_v,
         gsemA, gsemB, gsemC, ssemA, ssemB, ssemC, accA) = \
            refs[2 * rounds + 3:]

        c = lax.axis_index("c")
        s = lax.axis_index("s")
        base = c * HALF
        zvec = jnp.zeros((L,), jnp.float32)
        r0 = s * RPT

        for i in range(L):
            for k in range(128 // L):
                zero_v[i, pl.ds(k * L, L)] = zvec

        def zero_acc():
            for j in range(2 * RPT // L):
                pltpu.sync_copy(zero_v,
                                accA.at[pl.ds(2 * r0 + j * L, L)])

        zero_acc()

        for f in range(rounds):
            fs_h = fs_refs[f]
            plsc.subcore_barrier()

            def blk_body(blk, bcarry):
                e0 = s * EPT + blk * BLK
                pltpu.sync_copy(src_h.at[pl.ds(e0, BLK)], sblk)
                pltpu.sync_copy(dst_h.at[pl.ds(e0, BLK)], dblk)
                pltpu.sync_copy(ee_h.at[pl.ds(e0 * L, BLK * L)], eeblk)

                def gstart(chunk, rows_ref, gsem):
                    sl16 = pl.ds(chunk * L, L)
                    pltpu.async_copy(fs_h.at[sblk.at[sl16]], rows_ref, gsem)

                def gdrain(rows_ref, gsem):
                    pltpu.make_async_copy(fs_h.at[pl.ds(0, L)], rows_ref,
                                          gsem).wait()

                def sdrain(rL, rR, locL, locR, ssem):
                    pltpu.make_async_copy(rL, accA.at[locL], ssem).wait()
                    pltpu.make_async_copy(rR, accA.at[locR], ssem).wait()

                def process(chunk, rows_ref, rL, rR, locL, locR, gsem,
                            ssem):
                    dv = dblk[pl.ds(chunk * L, L)]
                    loc = dv - base
                    oob = (loc < 0) | (loc >= HALF)
                    loc = jnp.where(oob, HALF, loc)
                    locL[...] = loc * 2
                    locR[...] = loc * 2 + 1
                    gdrain(rows_ref, gsem)
                    for l in range(L):
                        eerow = eeblk[pl.ds((chunk * L + l) * L, L)]
                        scs = [eerow[f * hr + h] for h in range(hr)]
                        for k in range(RW // L):
                            sl = pl.ds((k % 8) * L, L)
                            half = rL if k < 8 else rR
                            half[l, sl] = (rows_ref[l, pl.ds(k * L, L)]
                                           * scs[k // vh])
                    pltpu.async_copy(rL, accA.at[locL], ssem, add=True)
                    pltpu.async_copy(rR, accA.at[locR], ssem, add=True)

                NT = (CPB - 1) // 3

                gstart(0, rowsA, gsemA)
                gstart(1, rowsB, gsemB)

                def triple(j, carry):
                    c0 = 3 * j

                    @pl.when(j > 0)
                    def _():
                        sdrain(rLC, rRC, locLC, locRC, ssemC)

                    gstart(c0 + 2, rowsC, gsemC)
                    process(c0, rowsA, rLA, rRA, locLA, locRA, gsemA,
                            ssemA)
                    process(c0 + 1, rowsB, rLB, rRB, locLB, locRB, gsemB,
                            ssemB)
                    sdrain(rLA, rRA, locLA, locRA, ssemA)
                    gstart(c0 + 3, rowsA, gsemA)
                    process(c0 + 2, rowsC, rLC, rRC, locLC, locRC, gsemC,
                            ssemC)
                    sdrain(rLB, rRB, locLB, locRB, ssemB)

                    @pl.when(j < NT - 1)
                    def _():
                        gstart(c0 + 4, rowsB, gsemB)

                    return carry

                lax.fori_loop(0, NT, triple, 0)
                sdrain(rLC, rRC, locLC, locRC, ssemC)
                process(CPB - 1, rowsA, rLA, rRA, locLA, locRA, gsemA,
                        ssemA)
                sdrain(rLA, rRA, locLA, locRA, ssemA)
                return bcarry

            lax.fori_loop(0, NBLK, blk_body, 0)

            plsc.subcore_barrier()
            pltpu.sync_copy(accA.at[pl.ds(2 * r0, 2 * RPT)],
                            outs[f].at[pl.ds(2 * (c * ACCR + r0),
                                             2 * RPT)])
            if f < rounds - 1:
                zero_acc()

    return pl.kernel(body, out_type=out_type, mesh=mesh,
                     scratch_types=scratch)


_score = _make_score_kernel()
_agg1 = _make_agg_kernel(2)
_agg2 = _make_agg_kernel(1)


def _k1_body(x_ref, ws_ref, wd_ref, al_ref, ar_ref,
             fa_ref, fb_ref, el_ref, er_ref):
    xb = x_ref[...]
    g = (lax.broadcasted_iota(jnp.int32, (4 * 128, 128), 0) // 128
         == lax.broadcasted_iota(jnp.int32, (4 * 128, 128), 1)
         ).astype(jnp.float32)
    fs = jnp.dot(xb, ws_ref[...], preferred_element_type=jnp.float32)
    fa_ref[...] = fs[:, 0:256]
    fb_ref[...] = fs[:, 256:512]
    el_ref[...] = jnp.dot(fs * al_ref[...], g,
                          preferred_element_type=jnp.float32)
    fd = jnp.dot(xb, wd_ref[...], preferred_element_type=jnp.float32)
    er_ref[...] = jnp.dot(fd * ar_ref[...], g,
                          preferred_element_type=jnp.float32)


def _k3_body(ra_ref, rb_ref, dna_ref, dnb_ref, ba_ref, bb_ref, wsa_ref,
             wsb_ref, wda_ref, wdb_ref, al_ref, ar_ref, fs_ref, el_ref,
             er_ref):
    d4 = dna_ref[...] + dnb_ref[...]
    rr = lax.broadcasted_iota(jnp.int32, (H, 256), 0)
    ch = lax.broadcasted_iota(jnp.int32, (H, 256), 1) // 128
    sa = jnp.dot(d4, (ch == rr).astype(jnp.float32),
                 preferred_element_type=jnp.float32) + EPS
    sb = jnp.dot(d4, (ch + 2 == rr).astype(jnp.float32),
                 preferred_element_type=jnp.float32) + EPS
    ha = jnp.maximum(ra_ref[...] / sa + ba_ref[...], 0.0)
    hb = jnp.maximum(rb_ref[...] / sb + bb_ref[...], 0.0)
    fs2 = (jnp.dot(ha, wsa_ref[...], preferred_element_type=jnp.float32)
           + jnp.dot(hb, wsb_ref[...], preferred_element_type=jnp.float32))
    fs_ref[...] = fs2
    g2 = (lax.broadcasted_iota(jnp.int32, (256, 128), 0) // 64
          == lax.broadcasted_iota(jnp.int32, (256, 128), 1)
          ).astype(jnp.float32)
    el_ref[...] = jnp.dot(fs2 * al_ref[...], g2,
                          preferred_element_type=jnp.float32)
    fd2 = (jnp.dot(ha, wda_ref[...], preferred_element_type=jnp.float32)
           + jnp.dot(hb, wdb_ref[...], preferred_element_type=jnp.float32))
    er_ref[...] = jnp.dot(fd2 * ar_ref[...], g2,
                          preferred_element_type=jnp.float32)


def _k5_body(rc_ref, dna_ref, dnb_ref, b2_ref, o_ref):
    d4 = dna_ref[...] + dnb_ref[...]
    rr = lax.broadcasted_iota(jnp.int32, (H, 256), 0)
    ch = lax.broadcasted_iota(jnp.int32, (H, 256), 1) // 64
    sc = jnp.dot(d4, (ch == rr).astype(jnp.float32),
                 preferred_element_type=jnp.float32) + EPS
    t = rc_ref[...] / sc + b2_ref[...]
    m = ((lax.broadcasted_iota(jnp.int32, (256, 64), 0) % 64
          == lax.broadcasted_iota(jnp.int32, (256, 64), 1))
         ).astype(jnp.float32) * 0.25
    o_ref[...] = jnp.dot(t, m, preferred_element_type=jnp.float32)


def _row_spec(w):
    return pl.BlockSpec((BN, w), lambda i: (i, 0))


def _full_spec(shape):
    return pl.BlockSpec(shape, lambda i: tuple(0 for _ in shape))


_k1 = pl.pallas_call(
    _k1_body,
    grid=(GRID,),
    in_specs=[_row_spec(256), _full_spec((256, 512)), _full_spec((256, 512)),
              _full_spec((1, 512)), _full_spec((1, 512))],
    out_specs=[_row_spec(RW), _row_spec(RW), _row_spec(128), _row_spec(128)],
    out_shape=[jax.ShapeDtypeStruct((N, RW), jnp.float32),
               jax.ShapeDtypeStruct((N, RW), jnp.float32),
               jax.ShapeDtypeStruct((N, 128), jnp.float32),
               jax.ShapeDtypeStruct((N, 128), jnp.float32)],
)

_k3 = pl.pallas_call(
    _k3_body,
    grid=(GRID,),
    in_specs=[_row_spec(RW), _row_spec(RW), _row_spec(H), _row_spec(H),
              _full_spec((1, RW)), _full_spec((1, RW)),
              _full_spec((RW, RW)), _full_spec((RW, RW)),
              _full_spec((RW, RW)), _full_spec((RW, RW)),
              _full_spec((1, RW)), _full_spec((1, RW))],
    out_specs=[_row_spec(RW), _row_spec(128), _row_spec(128)],
    out_shape=[jax.ShapeDtypeStruct((N, RW), jnp.float32),
               jax.ShapeDtypeStruct((N, 128), jnp.float32),
               jax.ShapeDtypeStruct((N, 128), jnp.float32)],
)

_k5 = pl.pallas_call(
    _k5_body,
    grid=(GRID,),
    in_specs=[_row_spec(RW), _row_spec(H), _row_spec(H),
              _full_spec((1, RW))],
    out_specs=_row_spec(64),
    out_shape=jax.ShapeDtypeStruct((N, 64), jnp.float32),
)


def _unpad(a):
    return jnp.concatenate([a[:HALF], a[ACCR:ACCR + HALF]], axis=0)


def _edge_phase(fs_parts, el, er, src, dst, agg):
    pad = jnp.zeros((L,), jnp.int32)
    srcp = jnp.concatenate([src, pad])
    dstp = jnp.concatenate([dst, pad])
    ee, den = _score(el, er, srcp, dstp)
    raws = agg(*fs_parts, ee, src, dst)
    if not isinstance(raws, (list, tuple)):
        raws = (raws,)
    dna = den[0:N, :H]
    dnb = den[NP:NP + N, :H]
    return [_unpad(r.reshape(NP, RW)) for r in raws], (dna, dnb)


def kernel(x, edge_index1, edge_index2, W_src1, W_dst1, attn_l1, attn_r1,
           bias1, W_src2, W_dst2, attn_l2, attn_r2, bias2):
    al1 = attn_l1.reshape(1, H * 128)
    ar1 = attn_r1.reshape(1, H * 128)
    fa1, fb1, el1, er1 = _k1(x, W_src1, W_dst1, al1, ar1)

    (ra, rb), (dn1a, dn1b) = _edge_phase((fa1, fb1), el1, er1,
                                         edge_index1[0], edge_index1[1],
                                         _agg1)

    ba = bias1[:RW].reshape(1, RW)
    bb = bias1[RW:].reshape(1, RW)
    al2 = attn_l2.reshape(1, H * 64)
    ar2 = attn_r2.reshape(1, H * 64)
    fs2, el2, er2 = _k3(ra, rb, dn1a, dn1b, ba, bb,
                        W_src2[:RW], W_src2[RW:],
                        W_dst2[:RW], W_dst2[RW:], al2, ar2)

    (rc,), (dn2a, dn2b) = _edge_phase((fs2,), el2, er2,
                                      edge_index2[0], edge_index2[1],
                                      _agg2)

    return _k5(rc, dn2a, dn2b, bias2.reshape(1, RW))
